# baseline (speedup 1.0000x reference)
import functools

import jax
import jax.numpy as jnp
from jax import lax
from jax.experimental import pallas as pl
from jax.experimental.pallas import tpu as pltpu
from jax.experimental.pallas import tpu_sc as plsc

_NC = 2
_NS = 16
_NW = _NC * _NS
_L = 16
_K = 16


def _sc_body(B, S, D, x_hbm, emb_hbm, out_hbm,
             xb0, xb1, xb2, xb3, ebuf,
             si0, si1, si2, si3, so0, so1, so2, so3):
    xb = (xb0, xb1, xb2, xb3)
    si = (si0, si1, si2, si3)
    so = (so0, so1, so2, so3)
    c = lax.axis_index("c")
    s = lax.axis_index("s")
    wid = s * _NC + c
    spw = S // _NW
    seq0 = wid * spw
    blk = _K * D
    n_seq_blocks = spw // _K

    def x_off(i, b):
        return (b * S + seq0 + i * _K) * D

    def start_in(i, b, slot):
        pltpu.async_copy(x_hbm.at[pl.ds(x_off(i, b), blk)], xb[slot], si[slot])

    def wait_in(slot):
        pltpu.make_async_copy(x_hbm.at[pl.ds(0, blk)], xb[slot], si[slot]).wait()

    def start_out(i, b, slot):
        pltpu.async_copy(xb[slot], out_hbm.at[pl.ds(x_off(i, b), blk)], so[slot])

    def drain_out(slot):
        pltpu.make_async_copy(xb[slot], out_hbm.at[pl.ds(0, blk)], so[slot]).wait()

    start_in(0, 0, 0)
    start_in(0, 1, 1)

    def outer(i, carry):
        for sl in range(4):
            wait_in(sl)
            if sl == 0:
                pltpu.sync_copy(emb_hbm.at[pl.ds((seq0 + i * _K) * D, blk)], ebuf)

            def add_vec(j, _sl=sl):
                sv = pl.ds(j, _L)
                xb[_sl][sv] = xb[_sl][sv] + ebuf[sv]

            plsc.parallel_loop(0, blk, _L, unroll=16)(add_vec)
            start_out(i, sl, sl)

            if sl < 2:
                ns = sl + 2

                @pl.when(i >= 1)
                def _():
                    drain_out(ns)

                start_in(i, ns, ns)
            else:
                ns = sl - 2

                @pl.when(i < n_seq_blocks - 1)
                def _():
                    drain_out(ns)
                    start_in(i + 1, ns, ns)
        return carry

    lax.fori_loop(0, n_seq_blocks, outer, None)

    for sl in range(4):
        drain_out(sl)


def kernel(x, emb):
    B, S, D = x.shape
    xf = x.reshape(B * S * D)
    ef = emb[:S].reshape(S * D)
    mesh = plsc.VectorSubcoreMesh(core_axis_name="c", subcore_axis_name="s")
    k = pl.kernel(
        functools.partial(_sc_body, B, S, D),
        out_type=jax.ShapeDtypeStruct((B * S * D,), jnp.float32),
        mesh=mesh,
        scratch_types=(
            [pltpu.VMEM((_K * D,), jnp.float32)] * 5
            + [pltpu.SemaphoreType.DMA] * 8
        ),
    )
    return k(xf, ef).reshape(B, S, D)


# speedup vs baseline: 4.3274x; 4.3274x over previous
import jax
import jax.numpy as jnp
from jax.experimental import pallas as pl


_BS = 512


def _add_kernel(x_ref, e_ref, o_ref):
    o_ref[...] = x_ref[...] + e_ref[...][None, :, :]


def kernel(x, emb):
    B, S, D = x.shape
    return pl.pallas_call(
        _add_kernel,
        grid=(S // _BS,),
        in_specs=[
            pl.BlockSpec((B, _BS, D), lambda s: (0, s, 0)),
            pl.BlockSpec((_BS, D), lambda s: (s, 0)),
        ],
        out_specs=pl.BlockSpec((B, _BS, D), lambda s: (0, s, 0)),
        out_shape=jax.ShapeDtypeStruct(x.shape, x.dtype),
    )(x, emb[:S])
